# baseline (speedup 1.0000x reference)
import functools

import jax
import jax.numpy as jnp
from jax import lax
from jax.experimental import pallas as pl
from jax.experimental.pallas import tpu as pltpu
from jax.experimental.pallas import tpu_sc as plsc


def kernel(indices, table):
    B, S = indices.shape
    V, D = table.shape
    N = B * S

    info = plsc.get_sparse_core_info()
    NC, NS = info.num_cores, info.num_subcores
    NW = NC * NS
    CHUNK = 64
    NBUF = 8
    K = NBUF // 2
    assert N % (NW * CHUNK) == 0
    n_chunks = N // (NW * CHUNK)
    assert n_chunks % NBUF == 0 and n_chunks >= 3 * NBUF

    idx3 = indices.reshape(NW, n_chunks, CHUNK)

    mesh = plsc.VectorSubcoreMesh(core_axis_name="c", subcore_axis_name="s")

    @functools.partial(
        pl.kernel,
        mesh=mesh,
        out_type=jax.ShapeDtypeStruct((N, D), jnp.float32),
        scratch_types=(
            [pltpu.VMEM((n_chunks, CHUNK), jnp.int32)]
            + [pltpu.VMEM((CHUNK, D), jnp.float32)] * NBUF
            + [pltpu.SemaphoreType.DMA] * (2 * NBUF)
        ),
    )
    def sc_gather(idx_hbm, table_hbm, out_hbm, idx_v, *bufs_and_sems):
        bufs = bufs_and_sems[:NBUF]
        gs = bufs_and_sems[NBUF:2 * NBUF]
        ws = bufs_and_sems[2 * NBUF:]
        wid = lax.axis_index("s") * NC + lax.axis_index("c")
        base = wid * (n_chunks * CHUNK)
        pltpu.sync_copy(idx_hbm.at[wid], idx_v)

        def gather(j, b):
            return pltpu.make_async_copy(
                table_hbm.at[idx_v.at[j]], bufs[b], gs[b])

        def write(j, b):
            return pltpu.make_async_copy(
                bufs[b], out_hbm.at[pl.ds(base + j * CHUNK, CHUNK)], ws[b])

        def step(j, b):
            write(j - K, (b + K) % NBUF).wait()
            gather(j + K, (b + K) % NBUF).start()
            gather(j, b).wait()
            write(j, b).start()

        for j in range(K):
            gather(j, j).start()
        for j in range(K):
            gather(j + K, (j + K) % NBUF).start()
            gather(j, j).wait()
            write(j, j).start()

        for j in range(K, NBUF):
            step(j, j)

        def body(g, carry):
            j0 = NBUF * g
            for b in range(NBUF):
                step(j0 + b, b)
            return carry

        lax.fori_loop(1, n_chunks // NBUF - 1, body, 0)

        for j in range(n_chunks - NBUF, n_chunks - K):
            step(j, j % NBUF)
        for j in range(n_chunks - K, n_chunks):
            gather(j, j % NBUF).wait()
            write(j, j % NBUF).start()
        for j in range(n_chunks - NBUF, n_chunks):
            write(j, j % NBUF).wait()

    out = sc_gather(idx3, table)
    return out.reshape(B, S, D)


# speedup vs baseline: 1.0084x; 1.0084x over previous
import functools

import jax
import jax.numpy as jnp
from jax import lax
from jax.experimental import pallas as pl
from jax.experimental.pallas import tpu as pltpu
from jax.experimental.pallas import tpu_sc as plsc


def kernel(indices, table):
    B, S = indices.shape
    V, D = table.shape
    N = B * S

    info = plsc.get_sparse_core_info()
    NC, NS = info.num_cores, info.num_subcores
    NW = NC * NS
    CHUNK = 128
    assert N % (NW * CHUNK * 2) == 0
    n_chunks = N // (NW * CHUNK)
    n2 = n_chunks // 2

    idx3 = indices.reshape(NW, n_chunks, CHUNK)

    mesh = plsc.VectorSubcoreMesh(core_axis_name="c", subcore_axis_name="s")

    @functools.partial(
        pl.kernel,
        mesh=mesh,
        out_type=jax.ShapeDtypeStruct((N, D), jnp.float32),
        scratch_types=[
            pltpu.VMEM((n_chunks, CHUNK), jnp.int32),
            pltpu.VMEM((CHUNK, D), jnp.float32),
            pltpu.VMEM((CHUNK, D), jnp.float32),
            pltpu.SemaphoreType.DMA,
            pltpu.SemaphoreType.DMA,
            pltpu.SemaphoreType.DMA,
            pltpu.SemaphoreType.DMA,
        ],
    )
    def sc_gather(idx_hbm, table_hbm, out_hbm, idx_v, rows0, rows1,
                  gsem0, gsem1, wsem0, wsem1):
        wid = lax.axis_index("s") * NC + lax.axis_index("c")
        base = wid * (n_chunks * CHUNK)
        pltpu.sync_copy(idx_hbm.at[wid], idx_v)

        def gather(j, buf, sem):
            return pltpu.make_async_copy(table_hbm.at[idx_v.at[j]], buf, sem)

        def write(j, buf, sem):
            return pltpu.make_async_copy(
                buf, out_hbm.at[pl.ds(base + j * CHUNK, CHUNK)], sem)

        gather(0, rows0, gsem0).start()
        gather(1, rows1, gsem1).start()
        gather(0, rows0, gsem0).wait()
        write(0, rows0, wsem0).start()
        write(0, rows0, wsem0).wait()
        gather(2, rows0, gsem0).start()
        gather(1, rows1, gsem1).wait()
        write(1, rows1, wsem1).start()

        def body(g, carry):
            j = 2 * g
            write(j - 1, rows1, wsem1).wait()
            gather(j + 1, rows1, gsem1).start()
            gather(j, rows0, gsem0).wait()
            write(j, rows0, wsem0).start()
            write(j, rows0, wsem0).wait()
            gather(j + 2, rows0, gsem0).start()
            gather(j + 1, rows1, gsem1).wait()
            write(j + 1, rows1, wsem1).start()
            return carry

        lax.fori_loop(1, n2 - 1, body, 0)

        j = n_chunks - 2
        write(j - 1, rows1, wsem1).wait()
        gather(j + 1, rows1, gsem1).start()
        gather(j, rows0, gsem0).wait()
        write(j, rows0, wsem0).start()
        write(j, rows0, wsem0).wait()
        gather(j + 1, rows1, gsem1).wait()
        write(j + 1, rows1, wsem1).start()
        write(j + 1, rows1, wsem1).wait()

    out = sc_gather(idx3, table)
    return out.reshape(B, S, D)


# speedup vs baseline: 1.0281x; 1.0195x over previous
import functools

import jax
import jax.numpy as jnp
from jax import lax
from jax.experimental import pallas as pl
from jax.experimental.pallas import tpu as pltpu
from jax.experimental.pallas import tpu_sc as plsc


def kernel(indices, table):
    B, S = indices.shape
    V, D = table.shape
    N = B * S

    info = plsc.get_sparse_core_info()
    NC, NS = info.num_cores, info.num_subcores
    NW = NC * NS
    CHUNK = 128
    assert N % (NW * CHUNK * 2) == 0
    n_chunks = N // (NW * CHUNK)
    n2 = n_chunks // 2

    idx3 = indices.reshape(NW, n_chunks, CHUNK)

    mesh = plsc.VectorSubcoreMesh(core_axis_name="c", subcore_axis_name="s")

    @functools.partial(
        pl.kernel,
        mesh=mesh,
        out_type=jax.ShapeDtypeStruct((N, D), jnp.float32),
        scratch_types=[
            pltpu.VMEM((n_chunks, CHUNK), jnp.int32),
            pltpu.VMEM((CHUNK, D), jnp.float32),
            pltpu.VMEM((CHUNK, D), jnp.float32),
            pltpu.VMEM_SHARED((NS, 2, CHUNK, D), jnp.float32),
            pltpu.SemaphoreType.DMA,
            pltpu.SemaphoreType.DMA,
            pltpu.SemaphoreType.DMA,
            pltpu.SemaphoreType.DMA,
            pltpu.SemaphoreType.DMA,
            pltpu.SemaphoreType.DMA,
        ],
    )
    def sc_gather(idx_hbm, table_hbm, out_hbm, idx_v, rows0, rows1, sp,
                  gsem0, gsem1, csem0, csem1, wsem0, wsem1):
        sid = lax.axis_index("s")
        wid = sid * NC + lax.axis_index("c")
        base = wid * (n_chunks * CHUNK)
        pltpu.sync_copy(idx_hbm.at[wid], idx_v)
        rows = (rows0, rows1)
        gsem = (gsem0, gsem1)
        csem = (csem0, csem1)
        wsem = (wsem0, wsem1)

        def gather(j, p):
            return pltpu.make_async_copy(
                table_hbm.at[idx_v.at[j]], rows[p], gsem[p])

        def xbar(p):
            return pltpu.make_async_copy(rows[p], sp.at[sid, p], csem[p])

        def write(j, p):
            return pltpu.make_async_copy(
                sp.at[sid, p], out_hbm.at[pl.ds(base + j * CHUNK, CHUNK)],
                wsem[p])

        def step(j, p):
            gather(j, p).wait()
            write(j - 2, p).wait()
            xbar(p).start()
            xbar(p).wait()
            gather(j + 2, p).start()
            write(j, p).start()

        gather(0, 0).start()
        gather(1, 1).start()
        gather(0, 0).wait()
        xbar(0).start()
        xbar(0).wait()
        gather(2, 0).start()
        write(0, 0).start()
        gather(1, 1).wait()
        xbar(1).start()
        xbar(1).wait()
        gather(3, 1).start()
        write(1, 1).start()

        def body(g, carry):
            j = 2 * g
            step(j, 0)
            step(j + 1, 1)
            return carry

        lax.fori_loop(1, n2 - 1, body, 0)

        j = n_chunks - 2
        gather(j, 0).wait()
        write(j - 2, 0).wait()
        xbar(0).start()
        xbar(0).wait()
        write(j, 0).start()
        gather(j + 1, 1).wait()
        write(j - 1, 1).wait()
        xbar(1).start()
        xbar(1).wait()
        write(j + 1, 1).start()
        write(j, 0).wait()
        write(j + 1, 1).wait()

    out = sc_gather(idx3, table)
    return out.reshape(B, S, D)


# speedup vs baseline: 1.0558x; 1.0270x over previous
import functools

import jax
import jax.numpy as jnp
from jax import lax
from jax.experimental import pallas as pl
from jax.experimental.pallas import tpu as pltpu
from jax.experimental.pallas import tpu_sc as plsc


def kernel(indices, table):
    B, S = indices.shape
    V, D = table.shape
    N = B * S

    info = plsc.get_sparse_core_info()
    NC, NS = info.num_cores, info.num_subcores
    NW = NC * NS
    CHUNK = 128
    NBUF = 4
    NSP = 2
    assert N % (NW * CHUNK) == 0
    n_chunks = N // (NW * CHUNK)
    assert n_chunks % NBUF == 0 and n_chunks >= 3 * NBUF

    idx3 = indices.reshape(NW, n_chunks, CHUNK)

    mesh = plsc.VectorSubcoreMesh(core_axis_name="c", subcore_axis_name="s")

    @functools.partial(
        pl.kernel,
        mesh=mesh,
        out_type=jax.ShapeDtypeStruct((N, D), jnp.float32),
        scratch_types=(
            [pltpu.VMEM((n_chunks, CHUNK), jnp.int32)]
            + [pltpu.VMEM((CHUNK, D), jnp.float32)] * NBUF
            + [pltpu.VMEM_SHARED((NS, NSP, CHUNK, D), jnp.float32)]
            + [pltpu.SemaphoreType.DMA] * (NBUF + 2 * NSP)
        ),
    )
    def sc_gather(idx_hbm, table_hbm, out_hbm, idx_v, *rest):
        rows = rest[:NBUF]
        sp = rest[NBUF]
        gsem = rest[NBUF + 1:2 * NBUF + 1]
        csem = rest[2 * NBUF + 1:2 * NBUF + 1 + NSP]
        wsem = rest[2 * NBUF + 1 + NSP:]
        sid = lax.axis_index("s")
        wid = sid * NC + lax.axis_index("c")
        base = wid * (n_chunks * CHUNK)
        pltpu.sync_copy(idx_hbm.at[wid], idx_v)

        def gather(j, b):
            return pltpu.make_async_copy(
                table_hbm.at[idx_v.at[j]], rows[b], gsem[b])

        def xbar(b, p):
            return pltpu.make_async_copy(rows[b], sp.at[sid, p], csem[p])

        def write(j, p):
            return pltpu.make_async_copy(
                sp.at[sid, p], out_hbm.at[pl.ds(base + j * CHUNK, CHUNK)],
                wsem[p])

        def step(j, b, p):
            gather(j, b).wait()
            write(j - NSP, p).wait()
            xbar(b, p).start()
            xbar(b, p).wait()
            write(j, p).start()
            gather(j + NBUF, b).start()

        for j in range(NBUF):
            gather(j, j).start()
        for j in range(NBUF):
            b, p = j, j % NSP
            gather(j, b).wait()
            if j >= NSP:
                write(j - NSP, p).wait()
            xbar(b, p).start()
            xbar(b, p).wait()
            write(j, p).start()
            gather(j + NBUF, b).start()

        def body(g, carry):
            j0 = NBUF * g
            for b in range(NBUF):
                step(j0 + b, b, b % NSP)
            return carry

        lax.fori_loop(1, n_chunks // NBUF - 1, body, 0)

        for j in range(n_chunks - NBUF, n_chunks):
            b, p = j % NBUF, j % NSP
            gather(j, b).wait()
            write(j - NSP, p).wait()
            xbar(b, p).start()
            xbar(b, p).wait()
            write(j, p).start()
        for j in range(n_chunks - NSP, n_chunks):
            write(j, j % NSP).wait()

    out = sc_gather(idx3, table)
    return out.reshape(B, S, D)
